# Initial kernel scaffold; baseline (speedup 1.0000x reference)
import jax
import jax.numpy as jnp
from jax.experimental import pallas as pl


def kernel(x, edge_index, params):
    raise NotImplementedError("write your pallas kernel here")


# speedup vs baseline: 12.4481x; 12.4481x over previous
import functools

import jax
import jax.numpy as jnp
from jax.experimental import pallas as pl
from jax.experimental.pallas import tpu as pltpu

NREAL = 307
NP = 320
T = 12
FC = 64
FT = 64
KCH = 3
P = 12

_INTERPRET = False

_BKEYS = ['u1', 'u2', 'u3', 'be', 've', 'w1', 'w2', 'w3', 'bs', 'vs',
          'cw', 'cb', 'tw', 'tb', 'rw', 'rb', 'lng', 'lnb']

_dot = functools.partial(jnp.dot, preferred_element_type=jnp.float32,
                         precision=jax.lax.Precision.HIGHEST)


def _dg_t(a, b):
    return jax.lax.dot_general(a, b, (((1,), (1,)), ((), ())),
                               preferred_element_type=jnp.float32,
                               precision=jax.lax.Precision.HIGHEST)


def _softmax0(m):
    mx = jnp.max(m, axis=0, keepdims=True)
    e = jnp.exp(m - mx)
    return e / jnp.sum(e, axis=0, keepdims=True)


def _graph_kernel(row_ref, col_ref, a_ref):
    row = row_ref[...]
    col = col_ref[...]
    maskf = (row != col).astype(jnp.float32)
    ep = row.shape[1]
    ion = jax.lax.broadcasted_iota(jnp.int32, (NP, ep), 0)
    rt = (row == ion).astype(jnp.float32)
    ct = (col == ion).astype(jnp.float32)
    deg = jnp.sum(rt * maskf, axis=1, keepdims=True)
    dinv = jnp.where(deg > 0, jax.lax.rsqrt(jnp.maximum(deg, 1.0)), 0.0)
    dr = jnp.sum(rt * dinv, axis=0, keepdims=True)
    dc = jnp.sum(ct * dinv, axis=0, keepdims=True)
    nv = -dr * dc * maskf
    a_ref[...] = _dg_t(ct * nv, rt)


def _block(Xc, F, r, A):
    u1X = _dot(r['u1'][...], Xc)
    LHS1 = jnp.concatenate([u1X[:, t * F:(t + 1) * F] for t in range(T)],
                           axis=0)
    LHS = _dot(LHS1, r['u2'][...])
    io_r = jax.lax.broadcasted_iota(jnp.int32, (F * T, T), 0)
    io_c = jax.lax.broadcasted_iota(jnp.int32, (F * T, T), 1)
    seg = io_r // F == io_c
    u3t = jnp.concatenate([r['u3'][...]] * T, axis=0)
    RHSm = _dot(Xc, jnp.where(seg, u3t, 0.0))
    P1 = _dot(LHS, RHSm)
    Emat = _dot(r['ve'][...], jax.nn.sigmoid(P1 + r['be'][...]))
    Et = _softmax0(Emat)

    ew = _dot(Et, r['w1'][...])
    if F == 1:
        XW = _dot(Xc, ew)
    else:
        XW = Xc[:, 0:F] * ew[0:1, 0:1]
        for s in range(1, T):
            XW = XW + Xc[:, s * F:(s + 1) * F] * ew[s:s + 1, 0:1]
    LHSs = _dot(XW, r['w2'][...])
    w3t = jnp.concatenate([r['w3'][...]] * T, axis=0)
    Cw3 = _dot(Xc, jnp.where(seg, w3t, 0.0))
    Rm = _dot(Cw3, Et)
    P2 = _dg_t(LHSs, Rm)
    Q = jax.nn.sigmoid(P2 + r['bs'][...])
    Spre = _dot(r['vs'][...], Q)
    io0 = jax.lax.broadcasted_iota(jnp.int32, (NP, NP), 0)
    io1 = jax.lax.broadcasted_iota(jnp.int32, (NP, NP), 1)
    S = _softmax0(jnp.where(io0 < NREAL, Spre, -1e9))
    d = jnp.sum(jnp.where(io0 == io1, S, 0.0), axis=1, keepdims=True)

    St = S.T
    Tx0 = d * Xc
    Tx1 = _dot(A * St, Tx0)
    Tx2 = 2.0 * _dot(A, Tx1) - Tx0
    cwr = r['cw'][...]
    cb = r['cb'][...]
    sg = []
    for t in range(T):
        sl = slice(t * F, (t + 1) * F)
        cat = jnp.concatenate([Tx0[:, sl], Tx1[:, sl], Tx2[:, sl]], axis=1)
        sg.append(jnp.maximum(_dot(cat, cwr) + cb, 0.0))

    wcat = jnp.concatenate([r['tw'][...], r['rw'][...]], axis=0)
    bias = r['tb'][...] + r['rb'][...]
    Z = jnp.zeros((NP, FC), jnp.float32)
    g = r['lng'][...]
    bb = r['lnb'][...]
    outs = []
    for t in range(T):
        left = sg[t - 1] if t > 0 else Z
        right = sg[t + 1] if t < T - 1 else Z
        cat4 = jnp.concatenate([left, sg[t], right,
                                Xc[:, t * F:(t + 1) * F]], axis=1)
        H = jnp.maximum(_dot(cat4, wcat) + bias, 0.0)
        mu = jnp.mean(H, axis=1, keepdims=True)
        xc = H - mu
        var = jnp.mean(xc * xc, axis=1, keepdims=True)
        outs.append(xc * jax.lax.rsqrt(var + 1e-5) * g + bb)
    return jnp.concatenate(outs, axis=1)


def _main_kernel(*refs):
    x_ref = refs[0]
    a_ref = refs[1]
    b0 = dict(zip(_BKEYS, refs[2:20]))
    b1 = dict(zip(_BKEYS, refs[20:38]))
    fw_ref, fb_ref = refs[38], refs[39]
    out_ref = refs[40]

    A = a_ref[...]
    X0 = x_ref[0]
    X1 = _block(X0, 1, b0, A)
    X2 = _block(X1, FT, b1, A)
    out = jnp.maximum(_dot(X2, fw_ref[...]) + fb_ref[...], 0.0)
    out_ref[0] = out


def _padn(a, axis):
    pad = [(0, 0)] * a.ndim
    pad[axis] = (0, NP - a.shape[axis])
    return jnp.pad(a, pad)


def _prep_block(p, F):
    return {
        'u1': _padn(p['U1'].reshape(1, -1), 1),
        'u2': _padn(p['U2'], 1),
        'u3': p['U3'].reshape(F, 1),
        'be': p['be'][0],
        've': p['Ve'],
        'w1': p['W1'].reshape(T, 1),
        'w2': p['W2'],
        'w3': p['W3'].reshape(F, 1),
        'bs': _padn(_padn(p['bs'][0], 0), 1),
        'vs': _padn(_padn(p['Vs'], 0), 1),
        'cw': p['cw'].reshape(KCH * F, FC),
        'cb': p['cb'].reshape(1, FC),
        'tw': jnp.transpose(p['tw'][:, :, 0, :], (2, 1, 0)).reshape(3 * FC, FT),
        'tb': p['tb'].reshape(1, FT),
        'rw': p['rw'][:, :, 0, 0].T,
        'rb': p['rb'].reshape(1, FT),
        'lng': p['lng'].reshape(1, FT),
        'lnb': p['lnb'].reshape(1, FT),
    }


def _full_spec(shape):
    nz = len(shape)
    return pl.BlockSpec(shape, lambda b, _n=nz: (0,) * _n)


def kernel(x, edge_index, params):
    B = x.shape[0]
    E = edge_index.shape[1]
    EP = ((E + 127) // 128) * 128
    x3 = _padn(x[:, :, 0, :], 1)
    rowp = jnp.pad(edge_index[0], (0, EP - E)).reshape(1, EP)
    colp = jnp.pad(edge_index[1], (0, EP - E)).reshape(1, EP)

    A = pl.pallas_call(
        _graph_kernel,
        out_shape=jax.ShapeDtypeStruct((NP, NP), jnp.float32),
        interpret=_INTERPRET,
    )(rowp, colp)

    b0 = _prep_block(params['b0'], 1)
    b1 = _prep_block(params['b1'], FT)
    fwr = jnp.transpose(params['fw'][:, :, 0, :], (1, 2, 0)).reshape(T * FT, P)
    fbr = params['fb'].reshape(1, P)
    flat = [b0[k] for k in _BKEYS] + [b1[k] for k in _BKEYS] + [fwr, fbr]

    in_specs = [pl.BlockSpec((1, NP, T), lambda b: (b, 0, 0)),
                _full_spec((NP, NP))]
    in_specs += [_full_spec(a.shape) for a in flat]

    outp = pl.pallas_call(
        _main_kernel,
        grid=(B,),
        in_specs=in_specs,
        out_specs=pl.BlockSpec((1, NP, P), lambda b: (b, 0, 0)),
        out_shape=jax.ShapeDtypeStruct((B, NP, P), jnp.float32),
        compiler_params=pltpu.CompilerParams(
            dimension_semantics=("arbitrary",)),
        interpret=_INTERPRET,
    )(x3, A, *flat)
    return outp[:, :NREAL, :]
